# baseline (speedup 1.0000x reference)
import functools

import jax
import jax.numpy as jnp
from jax import lax
from jax.experimental import pallas as pl
from jax.experimental.pallas import tpu as pltpu
from jax.experimental.pallas import tpu_sc as plsc

_NC = 2
_NS = 16
_NW = _NC * _NS
_CHUNK = 128
_NB = 5
_TR_BW = 4096


def _tr_body(x_ref, o_ref):
    y = x_ref[...].T.reshape(-1, 2, 64)
    o_ref[...] = jnp.concatenate([y[:, 0, :], y[:, 1, :]], axis=1)


@jax.jit
def _format_table(wt):
    d, v = wt.shape
    grid = pl.cdiv(v, _TR_BW)
    return pl.pallas_call(
        _tr_body,
        grid=(grid,),
        in_specs=[pl.BlockSpec((d, _TR_BW), lambda j: (0, j))],
        out_specs=pl.BlockSpec((_TR_BW // 2, 128), lambda j: (j, 0)),
        out_shape=jax.ShapeDtypeStruct((v // 2, 128), jnp.float32),
    )(wt)


@functools.partial(jax.jit, static_argnames=("n",))
def _gather_rows(ids_flat, w2, n):
    bpw = n // _NW
    nch = bpw // _CHUNK
    ngrp = nch // _NB
    mesh = plsc.VectorSubcoreMesh(core_axis_name="c", subcore_axis_name="s")

    @functools.partial(
        pl.kernel,
        out_type=jax.ShapeDtypeStruct((n, 128), jnp.float32),
        mesh=mesh,
        compiler_params=pltpu.CompilerParams(use_tc_tiling_on_sc=False),
        scratch_types=[
            pltpu.VMEM((bpw,), jnp.int32),
            pltpu.VMEM((_NB, _CHUNK, 128), jnp.float32),
            pltpu.SemaphoreType.DMA((_NB,)),
            pltpu.SemaphoreType.DMA((_NB,)),
        ],
    )
    def k(ids_hbm, w2_hbm, out_hbm, idx_v, rows_v, gsem, osem):
        wid = lax.axis_index("s") * _NC + lax.axis_index("c")
        base = wid * bpw
        pltpu.sync_copy(ids_hbm.at[pl.ds(base, bpw)], idx_v)

        def gather(j, b):
            return pltpu.make_async_copy(
                w2_hbm.at[idx_v.at[pl.ds(j * _CHUNK, _CHUNK)]],
                rows_v.at[b],
                gsem.at[b],
            )

        def outcp(j, b):
            return pltpu.make_async_copy(
                rows_v.at[b],
                out_hbm.at[pl.ds(base + j * _CHUNK, _CHUNK)],
                osem.at[b],
            )

        for b in range(_NB):
            gather(b, b).start()

        def body(g, carry):
            for b in range(_NB):
                j = g * _NB + b
                gather(j, b).wait()
                outcp(j, b).start()
                nj = j + _NB

                @pl.when(nj < nch)
                def _():
                    outcp(j, b).wait()
                    gather(nj, b).start()

            return carry

        lax.fori_loop(0, ngrp, body, 0)

        for b in range(_NB):
            outcp(nch - _NB + b, b).wait()

    return k(ids_flat, w2)


def _perm_body(x_ref, par_ref, o_ref):
    h = o_ref.shape[0]
    d = o_ref.shape[1]
    x = x_ref[...].reshape(-1, h, 128)
    par = par_ref[...]
    for hh in range(h):
        rows = x[:, hh, :]
        sel = jnp.where((par[:, hh] == 1)[:, None], rows[:, d:], rows[:, :d])
        o_ref[hh, :, :] = sel.T


@functools.partial(jax.jit, static_argnames=("b", "h", "d"))
def _emit_native(rows, par, b, h, d):
    bb = b // _NW
    return pl.pallas_call(
        _perm_body,
        grid=(_NW,),
        in_specs=[
            pl.BlockSpec((bb * h, 128), lambda j: (j, 0)),
            pl.BlockSpec((bb, h), lambda j: (j, 0)),
        ],
        out_specs=pl.BlockSpec((h, d, bb), lambda j: (0, 0, j)),
        out_shape=jax.ShapeDtypeStruct((h, d, b), jnp.float32),
    )(rows, par)


def kernel(input_ids, weight):
    b, h = input_ids.shape
    v, d = weight.shape
    w2 = _format_table(weight.T)
    ids_flat = input_ids.reshape(-1).astype(jnp.int32)
    rows = _gather_rows(ids_flat >> 1, w2, b * h)
    out_t = _emit_native(rows, input_ids.astype(jnp.int32) & 1, b, h, d)
    return jnp.transpose(out_t, (2, 0, 1))


# speedup vs baseline: 1.1547x; 1.1547x over previous
import functools

import jax
import jax.numpy as jnp
from jax import lax
from jax.experimental import pallas as pl
from jax.experimental.pallas import tpu as pltpu
from jax.experimental.pallas import tpu_sc as plsc

_NC = 2
_NS = 16
_NW = _NC * _NS
_CHUNK = 128
_NB = 5
_TR_BW = 4096


def _tr_body(x_ref, o_ref):
    y = x_ref[...].T.reshape(-1, 2, 64)
    o_ref[...] = jnp.concatenate([y[:, 0, :], y[:, 1, :]], axis=1)


@jax.jit
def _format_table(wt):
    d, v = wt.shape
    grid = pl.cdiv(v, _TR_BW)
    return pl.pallas_call(
        _tr_body,
        grid=(grid,),
        in_specs=[pl.BlockSpec((d, _TR_BW), lambda j: (0, j))],
        out_specs=pl.BlockSpec((_TR_BW // 2, 128), lambda j: (j, 0)),
        out_shape=jax.ShapeDtypeStruct((v // 2, 128), jnp.float32),
    )(wt)


@functools.partial(jax.jit, static_argnames=("n",))
def _gather_rows(ids_flat, w2v, n):
    bpw = n // _NW
    nch = bpw // _CHUNK
    ngrp = nch // _NB
    d = w2v.shape[1]
    mesh = plsc.VectorSubcoreMesh(core_axis_name="c", subcore_axis_name="s")

    @functools.partial(
        pl.kernel,
        out_type=jax.ShapeDtypeStruct((n, d), jnp.float32),
        mesh=mesh,
        compiler_params=pltpu.CompilerParams(use_tc_tiling_on_sc=False),
        scratch_types=[
            pltpu.VMEM((bpw,), jnp.int32),
            pltpu.VMEM((_NB, _CHUNK, d), jnp.float32),
            pltpu.SemaphoreType.DMA((_NB,)),
            pltpu.SemaphoreType.DMA((_NB,)),
        ],
    )
    def k(ids_hbm, w_hbm, out_hbm, idx_v, rows_v, gsem, osem):
        wid = lax.axis_index("s") * _NC + lax.axis_index("c")
        base = wid * bpw
        pltpu.sync_copy(ids_hbm.at[pl.ds(base, bpw)], idx_v)

        def gather(j, b):
            return pltpu.make_async_copy(
                w_hbm.at[idx_v.at[pl.ds(j * _CHUNK, _CHUNK)]],
                rows_v.at[b],
                gsem.at[b],
            )

        def outcp(j, b):
            return pltpu.make_async_copy(
                rows_v.at[b],
                out_hbm.at[pl.ds(base + j * _CHUNK, _CHUNK)],
                osem.at[b],
            )

        for b in range(_NB):
            gather(b, b).start()

        def body(g, carry):
            for b in range(_NB):
                j = g * _NB + b
                gather(j, b).wait()
                outcp(j, b).start()
                nj = j + _NB

                @pl.when(nj < nch)
                def _():
                    outcp(j, b).wait()
                    gather(nj, b).start()

            return carry

        lax.fori_loop(0, ngrp, body, 0)

        for b in range(_NB):
            outcp(nch - _NB + b, b).wait()

    return k(ids_flat, w2v)


def _perm_body(x_ref, o_ref):
    h = o_ref.shape[0]
    d = o_ref.shape[1]
    x = x_ref[...].reshape(-1, h // 2, 128)
    for hh in range(h):
        s = (hh % 2) * d
        o_ref[hh, :, :] = x[:, hh // 2, s : s + d].T


@functools.partial(jax.jit, static_argnames=("b", "h", "d"))
def _emit_native(rows2, b, h, d):
    bb = b // _NW
    return pl.pallas_call(
        _perm_body,
        grid=(_NW,),
        in_specs=[pl.BlockSpec((bb * h // 2, 128), lambda j: (j, 0))],
        out_specs=pl.BlockSpec((h, d, bb), lambda j: (0, 0, j)),
        out_shape=jax.ShapeDtypeStruct((h, d, b), jnp.float32),
    )(rows2)


def kernel(input_ids, weight):
    b, h = input_ids.shape
    v, d = weight.shape
    w2 = _format_table(weight.T)
    ids_flat = input_ids.reshape(-1).astype(jnp.int32)
    rows = _gather_rows(ids_flat, w2.reshape(v, d), b * h)
    out_t = _emit_native(rows.reshape((b * h) // 2, 2 * d), b, h, d)
    return jnp.transpose(out_t, (2, 0, 1))


# speedup vs baseline: 1.3514x; 1.1703x over previous
import functools

import jax
import jax.numpy as jnp
from jax import lax
from jax.experimental import pallas as pl
from jax.experimental.pallas import tpu as pltpu
from jax.experimental.pallas import tpu_sc as plsc

_NC = 2
_NS = 16
_NW = _NC * _NS
_CHUNK = 128
_NB = 5
_TR_BW = 4096


def _tr_body(x_ref, o_ref):
    x = x_ref[...]
    o_ref[...] = jnp.concatenate(
        [x.T, jnp.zeros((x.shape[1], 128 - x.shape[0]), x.dtype)], axis=1
    )


@jax.jit
def _format_table(wt):
    d, v = wt.shape
    grid = pl.cdiv(v, _TR_BW)
    return pl.pallas_call(
        _tr_body,
        grid=(grid,),
        in_specs=[pl.BlockSpec((d, _TR_BW), lambda j: (0, j))],
        out_specs=pl.BlockSpec((_TR_BW, 128), lambda j: (j, 0)),
        out_shape=jax.ShapeDtypeStruct((v, 128), jnp.float32),
    )(wt)


@functools.partial(jax.jit, static_argnames=("n",))
def _gather_rows(ids_flat, w2v, n):
    bpw = n // _NW
    nch = bpw // _CHUNK
    ngrp = nch // _NB
    d = w2v.shape[1]
    mesh = plsc.VectorSubcoreMesh(core_axis_name="c", subcore_axis_name="s")

    @functools.partial(
        pl.kernel,
        out_type=jax.ShapeDtypeStruct((n, d), jnp.float32),
        mesh=mesh,
        compiler_params=pltpu.CompilerParams(use_tc_tiling_on_sc=False),
        scratch_types=[
            pltpu.VMEM((bpw,), jnp.int32),
            pltpu.VMEM((_NB, _CHUNK, d), jnp.float32),
            pltpu.SemaphoreType.DMA((_NB,)),
            pltpu.SemaphoreType.DMA((_NB,)),
        ],
    )
    def k(ids_hbm, w_hbm, out_hbm, idx_v, rows_v, gsem, osem):
        wid = lax.axis_index("s") * _NC + lax.axis_index("c")
        base = wid * bpw
        pltpu.sync_copy(ids_hbm.at[pl.ds(base, bpw)], idx_v)

        def gather(j, b):
            return pltpu.make_async_copy(
                w_hbm.at[idx_v.at[pl.ds(j * _CHUNK, _CHUNK)]],
                rows_v.at[b],
                gsem.at[b],
            )

        def outcp(j, b):
            return pltpu.make_async_copy(
                rows_v.at[b],
                out_hbm.at[pl.ds(base + j * _CHUNK, _CHUNK)],
                osem.at[b],
            )

        for b in range(_NB):
            gather(b, b).start()

        def body(g, carry):
            for b in range(_NB):
                j = g * _NB + b
                gather(j, b).wait()
                outcp(j, b).start()
                nj = j + _NB

                @pl.when(nj < nch)
                def _():
                    outcp(j, b).wait()
                    gather(nj, b).start()

            return carry

        lax.fori_loop(0, ngrp, body, 0)

        for b in range(_NB):
            outcp(nch - _NB + b, b).wait()

    return k(ids_flat, w2v)


def _perm_body(x_ref, o_ref):
    h = o_ref.shape[0]
    d = o_ref.shape[1]
    x = x_ref[...].reshape(-1, h // 2, 128)
    for hh in range(h):
        s = (hh % 2) * d
        o_ref[hh, :, :] = x[:, hh // 2, s : s + d].T


@functools.partial(jax.jit, static_argnames=("b", "h", "d"))
def _emit_native(rows2, b, h, d):
    bb = b // _NW
    return pl.pallas_call(
        _perm_body,
        grid=(_NW,),
        in_specs=[pl.BlockSpec((bb * h // 2, 128), lambda j: (j, 0))],
        out_specs=pl.BlockSpec((h, d, bb), lambda j: (0, 0, j)),
        out_shape=jax.ShapeDtypeStruct((h, d, b), jnp.float32),
    )(rows2)


def kernel(input_ids, weight):
    b, h = input_ids.shape
    v, d = weight.shape
    w2 = _format_table(weight.T)
    ids_flat = input_ids.reshape(-1).astype(jnp.int32)
    rows = _gather_rows(ids_flat * 2, w2.reshape(2 * v, d), b * h)
    out_t = _emit_native(rows.reshape((b * h) // 2, 2 * d), b, h, d)
    return jnp.transpose(out_t, (2, 0, 1))


# speedup vs baseline: 1.6102x; 1.1915x over previous
import functools

import jax
import jax.numpy as jnp
from jax import lax
from jax.experimental import pallas as pl
from jax.experimental.pallas import tpu as pltpu
from jax.experimental.pallas import tpu_sc as plsc

_NC = 2
_NS = 16
_NW = _NC * _NS
_CHUNK = 128
_NB = 5
_TR_BW = 8192


def _tr_body(x_ref, o_ref):
    x = x_ref[...]
    o_ref[...] = jnp.concatenate(
        [x.T, jnp.zeros((x.shape[1], 128 - x.shape[0]), x.dtype)], axis=1
    )


@jax.jit
def _format_table(wt):
    d, v = wt.shape
    grid = pl.cdiv(v, _TR_BW)
    return pl.pallas_call(
        _tr_body,
        grid=(grid,),
        in_specs=[pl.BlockSpec((d, _TR_BW), lambda j: (0, j))],
        out_specs=pl.BlockSpec((_TR_BW, 128), lambda j: (j, 0)),
        out_shape=jax.ShapeDtypeStruct((v, 128), jnp.float32),
    )(wt)


@functools.partial(jax.jit, static_argnames=("n",))
def _gather_rows(ids_flat, w2v, n):
    bpw = n // _NW
    nch = bpw // _CHUNK
    ngrp = nch // _NB
    d = w2v.shape[1]
    mesh = plsc.VectorSubcoreMesh(core_axis_name="c", subcore_axis_name="s")

    @functools.partial(
        pl.kernel,
        out_type=jax.ShapeDtypeStruct((n, d), jnp.float32),
        mesh=mesh,
        compiler_params=pltpu.CompilerParams(use_tc_tiling_on_sc=False),
        scratch_types=[
            pltpu.VMEM((bpw,), jnp.int32),
            pltpu.VMEM((_NB, _CHUNK, d), jnp.float32),
            pltpu.SemaphoreType.DMA((_NB,)),
            pltpu.SemaphoreType.DMA((_NB,)),
        ],
    )
    def k(ids_hbm, w_hbm, out_hbm, idx_v, rows_v, gsem, osem):
        wid = lax.axis_index("s") * _NC + lax.axis_index("c")
        base = wid * bpw
        pltpu.sync_copy(ids_hbm.at[pl.ds(base, bpw)], idx_v)

        def gather(j, b):
            return pltpu.make_async_copy(
                w_hbm.at[idx_v.at[pl.ds(j * _CHUNK, _CHUNK)]],
                rows_v.at[b],
                gsem.at[b],
            )

        def outcp(j, b):
            return pltpu.make_async_copy(
                rows_v.at[b],
                out_hbm.at[pl.ds(base + j * _CHUNK, _CHUNK)],
                osem.at[b],
            )

        for b in range(_NB):
            gather(b, b).start()

        def body(g, carry):
            for b in range(_NB):
                j = g * _NB + b
                gather(j, b).wait()
                outcp(j, b).start()
                nj = j + _NB

                @pl.when(nj < nch)
                def _():
                    outcp(j, b).wait()
                    gather(nj, b).start()

            return carry

        lax.fori_loop(0, ngrp, body, 0)

        for b in range(_NB):
            outcp(nch - _NB + b, b).wait()

    return k(ids_flat, w2v)


def _perm_body(x_ref, o_ref):
    h = o_ref.shape[0]
    d = o_ref.shape[1]
    x = x_ref[...].reshape(-1, h // 2, 128)
    for hh in range(h):
        s = (hh % 2) * d
        o_ref[hh, :, :] = x[:, hh // 2, s : s + d].T


@functools.partial(jax.jit, static_argnames=("b", "h", "d"))
def _emit_native(rows2, b, h, d):
    bb = b // _NW
    return pl.pallas_call(
        _perm_body,
        grid=(_NW,),
        in_specs=[pl.BlockSpec((bb * h // 2, 128), lambda j: (j, 0))],
        out_specs=pl.BlockSpec((h, d, bb), lambda j: (0, 0, j)),
        out_shape=jax.ShapeDtypeStruct((h, d, b), jnp.float32),
    )(rows2)


def kernel(input_ids, weight):
    b, h = input_ids.shape
    v, d = weight.shape
    w2 = _format_table(weight.T)
    ids_flat = input_ids.reshape(-1).astype(jnp.int32)
    rows = _gather_rows(ids_flat * 2, w2.reshape(2 * v, d), b * h)
    out_t = _emit_native(rows.reshape((b * h) // 2, 2 * d), b, h, d)
    return jnp.transpose(out_t, (2, 0, 1))


# speedup vs baseline: 1.6894x; 1.0491x over previous
import functools

import jax
import jax.numpy as jnp
from jax import lax
from jax.experimental import pallas as pl
from jax.experimental.pallas import tpu as pltpu
from jax.experimental.pallas import tpu_sc as plsc

_NC = 2
_NS = 16
_NW = _NC * _NS
_CHUNK = 128
_NB = 5
_TR_BW = 16384


def _tr_body(x_ref, o_ref):
    x = x_ref[...]
    o_ref[...] = jnp.concatenate(
        [x.T, jnp.zeros((x.shape[1], 128 - x.shape[0]), x.dtype)], axis=1
    )


@jax.jit
def _format_table(wt):
    d, v = wt.shape
    grid = pl.cdiv(v, _TR_BW)
    return pl.pallas_call(
        _tr_body,
        grid=(grid,),
        in_specs=[pl.BlockSpec((d, _TR_BW), lambda j: (0, j))],
        out_specs=pl.BlockSpec((_TR_BW, 128), lambda j: (j, 0)),
        out_shape=jax.ShapeDtypeStruct((v, 128), jnp.float32),
    )(wt)


@functools.partial(jax.jit, static_argnames=("n",))
def _gather_rows(ids_flat, w2v, n):
    bpw = n // _NW
    nch = bpw // _CHUNK
    ngrp = nch // _NB
    d = w2v.shape[1]
    mesh = plsc.VectorSubcoreMesh(core_axis_name="c", subcore_axis_name="s")

    @functools.partial(
        pl.kernel,
        out_type=jax.ShapeDtypeStruct((n, d), jnp.float32),
        mesh=mesh,
        compiler_params=pltpu.CompilerParams(use_tc_tiling_on_sc=False),
        scratch_types=[
            pltpu.VMEM((bpw,), jnp.int32),
            pltpu.VMEM((_NB, _CHUNK, d), jnp.float32),
            pltpu.SemaphoreType.DMA((_NB,)),
            pltpu.SemaphoreType.DMA((_NB,)),
        ],
    )
    def k(ids_hbm, w_hbm, out_hbm, idx_v, rows_v, gsem, osem):
        wid = lax.axis_index("s") * _NC + lax.axis_index("c")
        base = wid * bpw
        pltpu.sync_copy(ids_hbm.at[pl.ds(base, bpw)], idx_v)

        def gather(j, b):
            return pltpu.make_async_copy(
                w_hbm.at[idx_v.at[pl.ds(j * _CHUNK, _CHUNK)]],
                rows_v.at[b],
                gsem.at[b],
            )

        def outcp(j, b):
            return pltpu.make_async_copy(
                rows_v.at[b],
                out_hbm.at[pl.ds(base + j * _CHUNK, _CHUNK)],
                osem.at[b],
            )

        for b in range(_NB):
            gather(b, b).start()

        def body(g, carry):
            for b in range(_NB):
                j = g * _NB + b
                gather(j, b).wait()
                outcp(j, b).start()
                nj = j + _NB

                @pl.when(nj < nch)
                def _():
                    outcp(j, b).wait()
                    gather(nj, b).start()

            return carry

        lax.fori_loop(0, ngrp, body, 0)

        for b in range(_NB):
            outcp(nch - _NB + b, b).wait()

    return k(ids_flat, w2v)


def _perm_body(x_ref, o_ref):
    h = o_ref.shape[0]
    d = o_ref.shape[1]
    x = x_ref[...].reshape(-1, h // 2, 128)
    for hh in range(h):
        s = (hh % 2) * d
        o_ref[hh, :, :] = x[:, hh // 2, s : s + d].T


@functools.partial(jax.jit, static_argnames=("b", "h", "d"))
def _emit_native(rows2, b, h, d):
    bb = b // _NW
    return pl.pallas_call(
        _perm_body,
        grid=(_NW,),
        in_specs=[pl.BlockSpec((bb * h // 2, 128), lambda j: (j, 0))],
        out_specs=pl.BlockSpec((h, d, bb), lambda j: (0, 0, j)),
        out_shape=jax.ShapeDtypeStruct((h, d, b), jnp.float32),
    )(rows2)


def kernel(input_ids, weight):
    b, h = input_ids.shape
    v, d = weight.shape
    w2 = _format_table(weight.T)
    ids_flat = input_ids.reshape(-1).astype(jnp.int32)
    rows = _gather_rows(ids_flat * 2, w2.reshape(2 * v, d), b * h)
    out_t = _emit_native(rows.reshape((b * h) // 2, 2 * d), b, h, d)
    return jnp.transpose(out_t, (2, 0, 1))


# speedup vs baseline: 1.7168x; 1.0162x over previous
import functools

import jax
import jax.numpy as jnp
from jax import lax
from jax.experimental import pallas as pl
from jax.experimental.pallas import tpu as pltpu
from jax.experimental.pallas import tpu_sc as plsc

_NC = 2
_NS = 16
_NW = _NC * _NS
_CHUNK = 128
_NB = 5
_TR_BW = 32768


def _tr_body(x_ref, o_ref):
    x = x_ref[...]
    o_ref[...] = jnp.concatenate(
        [x.T, jnp.zeros((x.shape[1], 128 - x.shape[0]), x.dtype)], axis=1
    )


@jax.jit
def _format_table(wt):
    d, v = wt.shape
    grid = pl.cdiv(v, _TR_BW)
    return pl.pallas_call(
        _tr_body,
        grid=(grid,),
        in_specs=[pl.BlockSpec((d, _TR_BW), lambda j: (0, j))],
        out_specs=pl.BlockSpec((_TR_BW, 128), lambda j: (j, 0)),
        out_shape=jax.ShapeDtypeStruct((v, 128), jnp.float32),
    )(wt)


@functools.partial(jax.jit, static_argnames=("n",))
def _gather_rows(ids_flat, w2v, n):
    bpw = n // _NW
    nch = bpw // _CHUNK
    ngrp = nch // _NB
    d = w2v.shape[1]
    mesh = plsc.VectorSubcoreMesh(core_axis_name="c", subcore_axis_name="s")

    @functools.partial(
        pl.kernel,
        out_type=jax.ShapeDtypeStruct((n, d), jnp.float32),
        mesh=mesh,
        compiler_params=pltpu.CompilerParams(use_tc_tiling_on_sc=False),
        scratch_types=[
            pltpu.VMEM((bpw,), jnp.int32),
            pltpu.VMEM((_NB, _CHUNK, d), jnp.float32),
            pltpu.SemaphoreType.DMA((_NB,)),
            pltpu.SemaphoreType.DMA((_NB,)),
        ],
    )
    def k(ids_hbm, w_hbm, out_hbm, idx_v, rows_v, gsem, osem):
        wid = lax.axis_index("s") * _NC + lax.axis_index("c")
        base = wid * bpw
        pltpu.sync_copy(ids_hbm.at[pl.ds(base, bpw)], idx_v)

        def gather(j, b):
            return pltpu.make_async_copy(
                w_hbm.at[idx_v.at[pl.ds(j * _CHUNK, _CHUNK)]],
                rows_v.at[b],
                gsem.at[b],
            )

        def outcp(j, b):
            return pltpu.make_async_copy(
                rows_v.at[b],
                out_hbm.at[pl.ds(base + j * _CHUNK, _CHUNK)],
                osem.at[b],
            )

        for b in range(_NB):
            gather(b, b).start()

        def body(g, carry):
            for b in range(_NB):
                j = g * _NB + b
                gather(j, b).wait()
                outcp(j, b).start()
                nj = j + _NB

                @pl.when(nj < nch)
                def _():
                    outcp(j, b).wait()
                    gather(nj, b).start()

            return carry

        lax.fori_loop(0, ngrp, body, 0)

        for b in range(_NB):
            outcp(nch - _NB + b, b).wait()

    return k(ids_flat, w2v)


def _perm_body(x_ref, o_ref):
    h = o_ref.shape[0]
    d = o_ref.shape[1]
    x = x_ref[...].reshape(-1, h // 2, 128)
    for hh in range(h):
        s = (hh % 2) * d
        o_ref[hh, :, :] = x[:, hh // 2, s : s + d].T


@functools.partial(jax.jit, static_argnames=("b", "h", "d"))
def _emit_native(rows2, b, h, d):
    bb = b // _NW
    return pl.pallas_call(
        _perm_body,
        grid=(_NW,),
        in_specs=[pl.BlockSpec((bb * h // 2, 128), lambda j: (j, 0))],
        out_specs=pl.BlockSpec((h, d, bb), lambda j: (0, 0, j)),
        out_shape=jax.ShapeDtypeStruct((h, d, b), jnp.float32),
    )(rows2)


def kernel(input_ids, weight):
    b, h = input_ids.shape
    v, d = weight.shape
    w2 = _format_table(weight.T)
    ids_flat = input_ids.reshape(-1).astype(jnp.int32)
    rows = _gather_rows(ids_flat * 2, w2.reshape(2 * v, d), b * h)
    out_t = _emit_native(rows.reshape((b * h) // 2, 2 * d), b, h, d)
    return jnp.transpose(out_t, (2, 0, 1))
